# baseline (speedup 1.0000x reference)
import functools

import jax
import jax.numpy as jnp
from jax import lax
from jax.experimental import pallas as pl
from jax.experimental.pallas import tpu as pltpu
from jax.experimental.pallas import tpu_sc as plsc

N_NODES = 10000
N_EDGES = 320000
D = 128
DH = D // 2
NC, NS = 2, 16
B = 128
NB = 159
EPT = NB * B
E_PAD = EPT * NS
NP = 10016
RPT = NP // NS
DW = 8
NSLOT = 3
DUMMY = N_NODES


def _sc_aggregate(h2, src0_r, src1_r, dst_r):
  mesh = plsc.VectorSubcoreMesh(core_axis_name="c", subcore_axis_name="s")

  @functools.partial(
      pl.kernel,
      out_type=(
          jax.ShapeDtypeStruct((NC, NP, DH), jnp.float32),
          jax.ShapeDtypeStruct((NC, NP, DW), jnp.float32),
      ),
      mesh=mesh,
      compiler_params=pltpu.CompilerParams(use_tc_tiling_on_sc=False),
      scratch_types=[
          pltpu.VMEM((NB, B), jnp.int32),
          pltpu.VMEM((NB, B), jnp.int32),
          pltpu.VMEM((NSLOT, B, DH // 2), jnp.int32),
          pltpu.VMEM((1, B, DH), jnp.float32),
          pltpu.VMEM((B, DH), jnp.float32),
          pltpu.VMEM((B, DW), jnp.float32),
          pltpu.VMEM((B, DW), jnp.float32),
          pltpu.VMEM_SHARED((NP, DH), jnp.float32),
          pltpu.VMEM_SHARED((NP, DW), jnp.float32),
      ] + [pltpu.SemaphoreType.DMA] * (3 * NSLOT),
  )
  def agg(h2_hbm, src0_hbm, src1_hbm, dst_hbm, zrow_hbm, zdeg_hbm, ones_hbm,
          acc_out, deg_out,
          src_v, dst_v, rowsb_v, rows_v, zrow_v, zdeg_v, ones_v, acc_sh, deg_sh,
          *sems):
    c = lax.axis_index("c")
    s = lax.axis_index("s")
    gsems = sems[0:NSLOT]
    ssems = sems[NSLOT:2 * NSLOT]
    dsems = sems[2 * NSLOT:3 * NSLOT]

    @pl.when(c == 0)
    def _():
      pltpu.sync_copy(src0_hbm.at[s], src_v)

    @pl.when(c == 1)
    def _():
      pltpu.sync_copy(src1_hbm.at[s], src_v)

    for b in range(NSLOT):
      pltpu.async_copy(h2_hbm.at[src_v.at[b]], rowsb_v.at[b], gsems[b])

    pltpu.sync_copy(dst_hbm.at[s], dst_v)
    pltpu.sync_copy(zrow_hbm, zrow_v)
    pltpu.sync_copy(zdeg_hbm, zdeg_v)
    pltpu.sync_copy(ones_hbm, ones_v)

    base = s * RPT
    for z in range(RPT // B):
      pltpu.sync_copy(zrow_v, acc_sh.at[pl.ds(base + z * B, B)])
      pltpu.sync_copy(zdeg_v, deg_sh.at[pl.ds(base + z * B, B)])
    rem = RPT % B
    if rem:
      pltpu.sync_copy(zrow_v.at[pl.ds(0, rem)],
                      acc_sh.at[pl.ds(base + (RPT // B) * B, rem)])
      pltpu.sync_copy(zdeg_v.at[pl.ds(0, rem)],
                      deg_sh.at[pl.ds(base + (RPT // B) * B, rem)])
    plsc.subcore_barrier()

    def body(i, carry):
      for b in range(NSLOT):
        jj = NSLOT * i + b
        pltpu.make_async_copy(
            h2_hbm.at[src_v.at[0]], rowsb_v.at[b], gsems[b]).wait()

        def conv(r8, carry2):
          for u in range(8):
            r = r8 * 8 + u
            for k in range(DH // 32):
              g = rowsb_v[b, r, pl.ds(16 * k, 16)]
              ev = lax.bitcast_convert_type(g << 16, jnp.float32)
              od = lax.bitcast_convert_type(
                  g & jnp.int32(-65536), jnp.float32)
              rows_v[0, r, pl.ds(32 * k, 16)] = ev
              rows_v[0, r, pl.ds(32 * k + 16, 16)] = od
          return carry2

        lax.fori_loop(0, B // 8, conv, 0)
        pltpu.async_copy(
            rows_v.at[0], acc_sh.at[dst_v.at[jj]], ssems[b], add=True)

        @pl.when(c == b % 2)
        def _():
          pltpu.async_copy(
              ones_v, deg_sh.at[dst_v.at[jj]], dsems[b], add=True)

        pltpu.make_async_copy(
            rows_v.at[0], acc_sh.at[dst_v.at[0]], ssems[b]).wait()

        @pl.when(c == b % 2)
        def _():
          pltpu.make_async_copy(
              ones_v, deg_sh.at[dst_v.at[0]], dsems[b]).wait()

        nxt = jnp.minimum(jj + NSLOT, NB - 1)
        pltpu.async_copy(h2_hbm.at[src_v.at[nxt]], rowsb_v.at[b], gsems[b])
      return carry

    lax.fori_loop(0, NB // NSLOT, body, 0)
    for b in range(NSLOT):
      pltpu.make_async_copy(
          h2_hbm.at[src_v.at[0]], rowsb_v.at[b], gsems[b]).wait()

    plsc.subcore_barrier()

    rb = s * RPT
    pltpu.sync_copy(acc_sh.at[pl.ds(rb, RPT)], acc_out.at[c, pl.ds(rb, RPT)])
    pltpu.sync_copy(deg_sh.at[pl.ds(rb, RPT)], deg_out.at[c, pl.ds(rb, RPT)])

  zrow = jnp.zeros((B, DH), jnp.float32)
  zdeg = jnp.zeros((B, DW), jnp.float32)
  ones = jnp.ones((B, DW), jnp.float32)
  return agg(h2, src0_r, src1_r, dst_r, zrow, zdeg, ones)


def _tc_dense(h, acc, deg, W_self, W_neigh, b2):
  R = 400

  def body(h_ref, a_ref, d_ref, ws_ref, wn_ref, b_ref, o_ref):
    hh = h_ref[...]
    accs = jnp.concatenate([a_ref[0], a_ref[1]], axis=-1)
    degc = jnp.maximum(d_ref[0, :, 0:1] + d_ref[1, :, 0:1], 1.0)
    hn = accs / degc
    rst = jnp.dot(hh, ws_ref[...], preferred_element_type=jnp.float32)
    rst = rst + jnp.dot(hn, wn_ref[...], preferred_element_type=jnp.float32)
    rst = rst + b_ref[...]
    rst = jnp.where(rst > 0, rst, rst * 0.01)
    o_ref[...] = hh + rst

  return pl.pallas_call(
      body,
      grid=(N_NODES // R,),
      in_specs=[
          pl.BlockSpec((R, D), lambda i: (i, 0)),
          pl.BlockSpec((NC, R, DH), lambda i: (0, i, 0)),
          pl.BlockSpec((NC, R, DW), lambda i: (0, i, 0)),
          pl.BlockSpec((D, D), lambda i: (0, 0)),
          pl.BlockSpec((D, D), lambda i: (0, 0)),
          pl.BlockSpec((1, D), lambda i: (0, 0)),
      ],
      out_specs=pl.BlockSpec((R, D), lambda i: (i, 0)),
      out_shape=jax.ShapeDtypeStruct((N_NODES, D), jnp.float32),
  )(h, acc, deg, W_self, W_neigh, b2)


@jax.jit
def _impl(h, edge_index, W_self, W_neigh, b):
  src = edge_index[0]
  dst = edge_index[1]
  pad = E_PAD - N_EDGES
  src_p = jnp.concatenate([src, jnp.zeros((pad,), jnp.int32)])
  dst_p = jnp.concatenate([dst, jnp.full((pad,), DUMMY, jnp.int32)])
  perm = jnp.array(
      [32 * (q // 32) + (q % 32) // 2 + 16 * ((q % 32) % 2)
       for q in range(DH)], dtype=jnp.int32)
  hbf = h.reshape(N_NODES, 2, DH)[:, :, perm].astype(jnp.bfloat16)
  h2 = jax.lax.bitcast_convert_type(
      hbf.reshape(2 * N_NODES, DH // 2, 2), jnp.int32)
  src0_r = (src_p * 2).reshape(NS, NB, B)
  src1_r = (src_p * 2 + 1).reshape(NS, NB, B)
  dst_r = dst_p.reshape(NS, NB, B)
  acc, deg = _sc_aggregate(h2, src0_r, src1_r, dst_r)
  return _tc_dense(h, acc, deg, W_self, W_neigh, b.reshape(1, D))


def kernel(h, edge_index, W_self, W_neigh, b):
  return _impl(h, edge_index, W_self, W_neigh, b)


# speedup vs baseline: 1.0229x; 1.0229x over previous
import functools

import jax
import jax.numpy as jnp
from jax import lax
from jax.experimental import pallas as pl
from jax.experimental.pallas import tpu as pltpu
from jax.experimental.pallas import tpu_sc as plsc

N_NODES = 10000
N_EDGES = 320000
D = 128
DH = D // 2
NC, NS = 2, 16
B = 128
NB = 160
EPT = NB * B
E_PAD = EPT * NS
NP = 10016
RPT = NP // NS
DW = 8
NSLOT = 2
DUMMY = N_NODES


def _sc_aggregate(h2, src0_r, src1_r, dst_r):
  mesh = plsc.VectorSubcoreMesh(core_axis_name="c", subcore_axis_name="s")

  @functools.partial(
      pl.kernel,
      out_type=(
          jax.ShapeDtypeStruct((NC, NP, DH), jnp.float32),
          jax.ShapeDtypeStruct((NC, NP, DW), jnp.float32),
      ),
      mesh=mesh,
      compiler_params=pltpu.CompilerParams(use_tc_tiling_on_sc=False),
      scratch_types=[
          pltpu.VMEM((NB, B), jnp.int32),
          pltpu.VMEM((NB, B), jnp.int32),
          pltpu.VMEM((NSLOT, B, DH // 2), jnp.int32),
          pltpu.VMEM((NSLOT, B, DH), jnp.float32),
          pltpu.VMEM((B, DH), jnp.float32),
          pltpu.VMEM((B, DW), jnp.float32),
          pltpu.VMEM((B, DW), jnp.float32),
          pltpu.VMEM_SHARED((NP, DH), jnp.float32),
          pltpu.VMEM_SHARED((NP, DW), jnp.float32),
      ] + [pltpu.SemaphoreType.DMA] * (3 * NSLOT),
  )
  def agg(h2_hbm, src0_hbm, src1_hbm, dst_hbm, zrow_hbm, zdeg_hbm, ones_hbm,
          acc_out, deg_out,
          src_v, dst_v, rowsb_v, rows_v, zrow_v, zdeg_v, ones_v, acc_sh, deg_sh,
          *sems):
    c = lax.axis_index("c")
    s = lax.axis_index("s")
    gsems = sems[0:NSLOT]
    ssems = sems[NSLOT:2 * NSLOT]
    dsems = sems[2 * NSLOT:3 * NSLOT]

    @pl.when(c == 0)
    def _():
      pltpu.sync_copy(src0_hbm.at[s], src_v)

    @pl.when(c == 1)
    def _():
      pltpu.sync_copy(src1_hbm.at[s], src_v)

    for b in range(NSLOT):
      pltpu.async_copy(h2_hbm.at[src_v.at[b]], rowsb_v.at[b], gsems[b])

    pltpu.sync_copy(dst_hbm.at[s], dst_v)
    pltpu.sync_copy(zrow_hbm, zrow_v)
    pltpu.sync_copy(zdeg_hbm, zdeg_v)
    pltpu.sync_copy(ones_hbm, ones_v)

    base = s * RPT
    for z in range(RPT // B):
      pltpu.sync_copy(zrow_v, acc_sh.at[pl.ds(base + z * B, B)])
      pltpu.sync_copy(zdeg_v, deg_sh.at[pl.ds(base + z * B, B)])
    rem = RPT % B
    if rem:
      pltpu.sync_copy(zrow_v.at[pl.ds(0, rem)],
                      acc_sh.at[pl.ds(base + (RPT // B) * B, rem)])
      pltpu.sync_copy(zdeg_v.at[pl.ds(0, rem)],
                      deg_sh.at[pl.ds(base + (RPT // B) * B, rem)])
    plsc.subcore_barrier()

    def body(i, carry):
      for b in range(NSLOT):
        jj = NSLOT * i + b
        pltpu.make_async_copy(
            h2_hbm.at[src_v.at[0]], rowsb_v.at[b], gsems[b]).wait()

        @pl.when(i > 0)
        def _():
          pltpu.make_async_copy(
              rows_v.at[b], acc_sh.at[dst_v.at[0]], ssems[b]).wait()

          @pl.when(c == b % 2)
          def _():
            pltpu.make_async_copy(
                ones_v, deg_sh.at[dst_v.at[0]], dsems[b]).wait()

        def conv(r8, carry2):
          for u in range(8):
            r = r8 * 8 + u
            for k in range(DH // 32):
              g = rowsb_v[b, r, pl.ds(16 * k, 16)]
              ev = lax.bitcast_convert_type(g << 16, jnp.float32)
              od = lax.bitcast_convert_type(
                  g & jnp.int32(-65536), jnp.float32)
              rows_v[b, r, pl.ds(32 * k, 16)] = ev
              rows_v[b, r, pl.ds(32 * k + 16, 16)] = od
          return carry2

        lax.fori_loop(0, B // 8, conv, 0)
        pltpu.async_copy(
            rows_v.at[b], acc_sh.at[dst_v.at[jj]], ssems[b], add=True)

        @pl.when(c == b % 2)
        def _():
          pltpu.async_copy(
              ones_v, deg_sh.at[dst_v.at[jj]], dsems[b], add=True)

        nxt = jnp.minimum(jj + NSLOT, NB - 1)
        pltpu.async_copy(h2_hbm.at[src_v.at[nxt]], rowsb_v.at[b], gsems[b])
      return carry

    lax.fori_loop(0, NB // NSLOT, body, 0)
    for b in range(NSLOT):
      pltpu.make_async_copy(
          rows_v.at[b], acc_sh.at[dst_v.at[0]], ssems[b]).wait()

      @pl.when(c == b % 2)
      def _():
        pltpu.make_async_copy(
            ones_v, deg_sh.at[dst_v.at[0]], dsems[b]).wait()

      pltpu.make_async_copy(
          h2_hbm.at[src_v.at[0]], rowsb_v.at[b], gsems[b]).wait()

    plsc.subcore_barrier()

    rb = s * RPT
    pltpu.sync_copy(acc_sh.at[pl.ds(rb, RPT)], acc_out.at[c, pl.ds(rb, RPT)])
    pltpu.sync_copy(deg_sh.at[pl.ds(rb, RPT)], deg_out.at[c, pl.ds(rb, RPT)])

  zrow = jnp.zeros((B, DH), jnp.float32)
  zdeg = jnp.zeros((B, DW), jnp.float32)
  ones = jnp.ones((B, DW), jnp.float32)
  return agg(h2, src0_r, src1_r, dst_r, zrow, zdeg, ones)


def _tc_dense(h, acc, deg, W_self, W_neigh, b2):
  R = 400

  def body(h_ref, a_ref, d_ref, ws_ref, wn_ref, b_ref, o_ref):
    hh = h_ref[...]
    accs = jnp.concatenate([a_ref[0], a_ref[1]], axis=-1)
    degc = jnp.maximum(d_ref[0, :, 0:1] + d_ref[1, :, 0:1], 1.0)
    hn = accs / degc
    rst = jnp.dot(hh, ws_ref[...], preferred_element_type=jnp.float32)
    rst = rst + jnp.dot(hn, wn_ref[...], preferred_element_type=jnp.float32)
    rst = rst + b_ref[...]
    rst = jnp.where(rst > 0, rst, rst * 0.01)
    o_ref[...] = hh + rst

  return pl.pallas_call(
      body,
      grid=(N_NODES // R,),
      in_specs=[
          pl.BlockSpec((R, D), lambda i: (i, 0)),
          pl.BlockSpec((NC, R, DH), lambda i: (0, i, 0)),
          pl.BlockSpec((NC, R, DW), lambda i: (0, i, 0)),
          pl.BlockSpec((D, D), lambda i: (0, 0)),
          pl.BlockSpec((D, D), lambda i: (0, 0)),
          pl.BlockSpec((1, D), lambda i: (0, 0)),
      ],
      out_specs=pl.BlockSpec((R, D), lambda i: (i, 0)),
      out_shape=jax.ShapeDtypeStruct((N_NODES, D), jnp.float32),
  )(h, acc, deg, W_self, W_neigh, b2)


@jax.jit
def _impl(h, edge_index, W_self, W_neigh, b):
  src = edge_index[0]
  dst = edge_index[1]
  pad = E_PAD - N_EDGES
  src_p = jnp.concatenate([src, jnp.zeros((pad,), jnp.int32)])
  dst_p = jnp.concatenate([dst, jnp.full((pad,), DUMMY, jnp.int32)])
  perm = jnp.array(
      [32 * (q // 32) + (q % 32) // 2 + 16 * ((q % 32) % 2)
       for q in range(DH)], dtype=jnp.int32)
  hbf = h.reshape(N_NODES, 2, DH)[:, :, perm].astype(jnp.bfloat16)
  h2 = jax.lax.bitcast_convert_type(
      hbf.reshape(2 * N_NODES, DH // 2, 2), jnp.int32)
  src0_r = (src_p * 2).reshape(NS, NB, B)
  src1_r = (src_p * 2 + 1).reshape(NS, NB, B)
  dst_r = dst_p.reshape(NS, NB, B)
  acc, deg = _sc_aggregate(h2, src0_r, src1_r, dst_r)
  return _tc_dense(h, acc, deg, W_self, W_neigh, b.reshape(1, D))


def kernel(h, edge_index, W_self, W_neigh, b):
  return _impl(h, edge_index, W_self, W_neigh, b)


# speedup vs baseline: 2.1515x; 2.1034x over previous
import functools

import jax
import jax.numpy as jnp
from jax import lax
from jax.experimental import pallas as pl
from jax.experimental.pallas import tpu as pltpu
from jax.experimental.pallas import tpu_sc as plsc

N_NODES = 10000
N_EDGES = 320000
D = 128
DH = D // 2
NC, NS = 2, 16
B = 128
NB = 159
EPT = NB * B
E_PAD = EPT * NS
NP = 10240
RPT = NP // NS
DW = 8
NSLOT = 3
DUMMY = N_NODES


def _sc_aggregate(h2, src0_r, src1_r, dst_r):
  mesh = plsc.VectorSubcoreMesh(core_axis_name="c", subcore_axis_name="s")

  @functools.partial(
      pl.kernel,
      out_type=(
          jax.ShapeDtypeStruct((NC, NP, DH), jnp.float32),
          jax.ShapeDtypeStruct((NC, NP, DW), jnp.float32),
      ),
      mesh=mesh,
      compiler_params=pltpu.CompilerParams(use_tc_tiling_on_sc=False),
      scratch_types=[
          pltpu.VMEM((NB, B), jnp.int32),
          pltpu.VMEM((NB, B), jnp.int32),
          pltpu.VMEM((NSLOT, B, DH), jnp.float32),
          pltpu.VMEM((B, DH), jnp.float32),
          pltpu.VMEM((B, DW), jnp.float32),
          pltpu.VMEM((B, DW), jnp.float32),
          pltpu.VMEM_SHARED((NP, DH), jnp.float32),
          pltpu.VMEM_SHARED((NP, DW), jnp.float32),
      ] + [pltpu.SemaphoreType.DMA] * (3 * NSLOT),
  )
  def agg(h2_hbm, src0_hbm, src1_hbm, dst_hbm, zrow_hbm, zdeg_hbm, ones_hbm,
          acc_out, deg_out,
          src_v, dst_v, rows_v, zrow_v, zdeg_v, ones_v, acc_sh, deg_sh,
          *sems):
    c = lax.axis_index("c")
    s = lax.axis_index("s")
    gsems = sems[0:NSLOT]
    ssems = sems[NSLOT:2 * NSLOT]
    dsems = sems[2 * NSLOT:3 * NSLOT]

    @pl.when(c == 0)
    def _():
      pltpu.sync_copy(src0_hbm.at[s], src_v)

    @pl.when(c == 1)
    def _():
      pltpu.sync_copy(src1_hbm.at[s], src_v)

    for b in range(NSLOT):
      pltpu.async_copy(h2_hbm.at[src_v.at[b]], rows_v.at[b], gsems[b])

    pltpu.sync_copy(dst_hbm.at[s], dst_v)
    pltpu.sync_copy(zrow_hbm, zrow_v)
    pltpu.sync_copy(zdeg_hbm, zdeg_v)
    pltpu.sync_copy(ones_hbm, ones_v)

    base = s * RPT
    for z in range(RPT // B):
      pltpu.sync_copy(zrow_v, acc_sh.at[pl.ds(base + z * B, B)])
      pltpu.sync_copy(zdeg_v, deg_sh.at[pl.ds(base + z * B, B)])
    plsc.subcore_barrier()

    def body(i, carry):
      for b in range(NSLOT):
        jj = NSLOT * i + b
        pltpu.make_async_copy(
            h2_hbm.at[src_v.at[0]], rows_v.at[b], gsems[b]).wait()

        @pl.when(i > 0)
        def _():
          pltpu.make_async_copy(
              rows_v.at[b], acc_sh.at[dst_v.at[0]], ssems[b]).wait()

          @pl.when(c == b % 2)
          def _():
            pltpu.make_async_copy(
                ones_v, deg_sh.at[dst_v.at[0]], dsems[b]).wait()

        pltpu.async_copy(
            rows_v.at[b], acc_sh.at[dst_v.at[jj]], ssems[b], add=True)

        @pl.when(c == b % 2)
        def _():
          pltpu.async_copy(
              ones_v, deg_sh.at[dst_v.at[jj]], dsems[b], add=True)

        nxt = jnp.minimum(jj + NSLOT, NB - 1)
        pltpu.async_copy(h2_hbm.at[src_v.at[nxt]], rows_v.at[b], gsems[b])
      return carry

    lax.fori_loop(0, NB // NSLOT, body, 0)
    for b in range(NSLOT):
      pltpu.make_async_copy(
          rows_v.at[b], acc_sh.at[dst_v.at[0]], ssems[b]).wait()

      @pl.when(c == b % 2)
      def _():
        pltpu.make_async_copy(
            ones_v, deg_sh.at[dst_v.at[0]], dsems[b]).wait()

      pltpu.make_async_copy(
          h2_hbm.at[src_v.at[0]], rows_v.at[b], gsems[b]).wait()

    plsc.subcore_barrier()

    rb = s * RPT
    pltpu.sync_copy(acc_sh.at[pl.ds(rb, RPT)], acc_out.at[c, pl.ds(rb, RPT)])
    pltpu.sync_copy(deg_sh.at[pl.ds(rb, RPT)], deg_out.at[c, pl.ds(rb, RPT)])

  zrow = jnp.zeros((B, DH), jnp.float32)
  zdeg = jnp.zeros((B, DW), jnp.float32)
  ones = jnp.ones((B, DW), jnp.float32)
  return agg(h2, src0_r, src1_r, dst_r, zrow, zdeg, ones)


def _tc_dense(h, acc, deg, W_self, W_neigh, b2):
  R = 400

  def body(h_ref, a_ref, d_ref, ws_ref, wn_ref, b_ref, o_ref):
    hh = h_ref[...]
    accs = jnp.concatenate([a_ref[0], a_ref[1]], axis=-1)
    degc = jnp.maximum(d_ref[0, :, 0:1] + d_ref[1, :, 0:1], 1.0)
    hn = accs / degc
    rst = jnp.dot(hh, ws_ref[...], preferred_element_type=jnp.float32)
    rst = rst + jnp.dot(hn, wn_ref[...], preferred_element_type=jnp.float32)
    rst = rst + b_ref[...]
    rst = jnp.where(rst > 0, rst, rst * 0.01)
    o_ref[...] = hh + rst

  return pl.pallas_call(
      body,
      grid=(N_NODES // R,),
      in_specs=[
          pl.BlockSpec((R, D), lambda i: (i, 0)),
          pl.BlockSpec((NC, R, DH), lambda i: (0, i, 0)),
          pl.BlockSpec((NC, R, DW), lambda i: (0, i, 0)),
          pl.BlockSpec((D, D), lambda i: (0, 0)),
          pl.BlockSpec((D, D), lambda i: (0, 0)),
          pl.BlockSpec((1, D), lambda i: (0, 0)),
      ],
      out_specs=pl.BlockSpec((R, D), lambda i: (i, 0)),
      out_shape=jax.ShapeDtypeStruct((N_NODES, D), jnp.float32),
  )(h, acc, deg, W_self, W_neigh, b2)


@jax.jit
def _impl(h, edge_index, W_self, W_neigh, b):
  src = edge_index[0]
  dst = edge_index[1]
  pad = E_PAD - N_EDGES
  src_p = jnp.concatenate([src, jnp.zeros((pad,), jnp.int32)])
  dst_p = jnp.concatenate([dst, jnp.full((pad,), DUMMY, jnp.int32)])
  h2 = h.reshape(2 * N_NODES, DH)
  src0_r = (src_p * 2).reshape(NS, NB, B)
  src1_r = (src_p * 2 + 1).reshape(NS, NB, B)
  dst_r = dst_p.reshape(NS, NB, B)
  acc, deg = _sc_aggregate(h2, src0_r, src1_r, dst_r)
  return _tc_dense(h, acc, deg, W_self, W_neigh, b.reshape(1, D))


def kernel(h, edge_index, W_self, W_neigh, b):
  return _impl(h, edge_index, W_self, W_neigh, b)


# speedup vs baseline: 2.1871x; 1.0165x over previous
import functools

import jax
import jax.numpy as jnp
from jax import lax
from jax.experimental import pallas as pl
from jax.experimental.pallas import tpu as pltpu
from jax.experimental.pallas import tpu_sc as plsc

N_NODES = 10000
N_EDGES = 320000
D = 128
DH = D // 2
NC, NS = 2, 16
B = 128
NB = 159
EPT = NB * B
E_PAD = EPT * NS
NP = 10240
RPT = NP // NS
DW = 8
NSLOT = 3
DUMMY = N_NODES


def _sc_aggregate(h2, src0_r, src1_r, dst_r):
  mesh = plsc.VectorSubcoreMesh(core_axis_name="c", subcore_axis_name="s")

  @functools.partial(
      pl.kernel,
      out_type=(
          jax.ShapeDtypeStruct((NC, NP, DH), jnp.float32),
          jax.ShapeDtypeStruct((NC, NP, DW), jnp.float32),
      ),
      mesh=mesh,
      compiler_params=pltpu.CompilerParams(use_tc_tiling_on_sc=False),
      scratch_types=[
          pltpu.VMEM((NB, B), jnp.int32),
          pltpu.VMEM((NB, B), jnp.int32),
          pltpu.VMEM((NSLOT, B, DH), jnp.float32),
          pltpu.VMEM((B, DH), jnp.float32),
          pltpu.VMEM((B, DW), jnp.float32),
          pltpu.VMEM((B, DW), jnp.float32),
          pltpu.VMEM_SHARED((NP, DH), jnp.float32),
          pltpu.VMEM_SHARED((NP, DW), jnp.float32),
      ] + [pltpu.SemaphoreType.DMA] * (3 * NSLOT),
  )
  def agg(h2_hbm, src0_hbm, src1_hbm, dst_hbm, zrow_hbm, zdeg_hbm, ones_hbm,
          acc_out, deg_out,
          src_v, dst_v, rows_v, zrow_v, zdeg_v, ones_v, acc_sh, deg_sh,
          *sems):
    c = lax.axis_index("c")
    s = lax.axis_index("s")
    gsems = sems[0:NSLOT]
    ssems = sems[NSLOT:2 * NSLOT]
    dsems = sems[2 * NSLOT:3 * NSLOT]

    @pl.when(c == 0)
    def _():
      pltpu.sync_copy(src0_hbm.at[s], src_v)

    @pl.when(c == 1)
    def _():
      pltpu.sync_copy(src1_hbm.at[s], src_v)

    for b in range(NSLOT):
      pltpu.async_copy(h2_hbm.at[src_v.at[b]], rows_v.at[b], gsems[b])

    pltpu.sync_copy(dst_hbm.at[s], dst_v)
    pltpu.sync_copy(zrow_hbm, zrow_v)
    pltpu.sync_copy(zdeg_hbm, zdeg_v)
    pltpu.sync_copy(ones_hbm, ones_v)

    base = s * RPT
    for z in range(RPT // B):
      pltpu.sync_copy(zrow_v, acc_sh.at[pl.ds(base + z * B, B)])
      pltpu.sync_copy(zdeg_v, deg_sh.at[pl.ds(base + z * B, B)])
    plsc.subcore_barrier()

    def body(i, carry):
      for b in range(NSLOT):
        jj = NSLOT * i + b
        pltpu.make_async_copy(
            h2_hbm.at[src_v.at[0]], rows_v.at[b], gsems[b]).wait()
        pltpu.async_copy(
            rows_v.at[b], acc_sh.at[dst_v.at[jj]], ssems[b], add=True)

        @pl.when(lax.rem(jj, 2) == c)
        def _():
          pltpu.async_copy(
              ones_v, deg_sh.at[dst_v.at[jj]], dsems[b], add=True)

        pltpu.make_async_copy(
            rows_v.at[b], acc_sh.at[dst_v.at[0]], ssems[b]).wait()

        @pl.when(lax.rem(jj, 2) == c)
        def _():
          pltpu.make_async_copy(
              ones_v, deg_sh.at[dst_v.at[0]], dsems[b]).wait()

        nxt = jnp.minimum(jj + NSLOT, NB - 1)
        pltpu.async_copy(h2_hbm.at[src_v.at[nxt]], rows_v.at[b], gsems[b])
      return carry

    lax.fori_loop(0, NB // NSLOT, body, 0)
    for b in range(NSLOT):
      pltpu.make_async_copy(
          h2_hbm.at[src_v.at[0]], rows_v.at[b], gsems[b]).wait()

    plsc.subcore_barrier()

    rb = s * RPT
    pltpu.sync_copy(acc_sh.at[pl.ds(rb, RPT)], acc_out.at[c, pl.ds(rb, RPT)])
    pltpu.sync_copy(deg_sh.at[pl.ds(rb, RPT)], deg_out.at[c, pl.ds(rb, RPT)])

  zrow = jnp.zeros((B, DH), jnp.float32)
  zdeg = jnp.zeros((B, DW), jnp.float32)
  ones = jnp.ones((B, DW), jnp.float32)
  return agg(h2, src0_r, src1_r, dst_r, zrow, zdeg, ones)


def _tc_dense(h, acc, deg, W_self, W_neigh, b2):
  R = 400

  def body(h_ref, a_ref, d_ref, ws_ref, wn_ref, b_ref, o_ref):
    hh = h_ref[...]
    accs = jnp.concatenate([a_ref[0], a_ref[1]], axis=-1)
    degc = jnp.maximum(d_ref[0, :, 0:1] + d_ref[1, :, 0:1], 1.0)
    hn = accs / degc
    rst = jnp.dot(hh, ws_ref[...], preferred_element_type=jnp.float32)
    rst = rst + jnp.dot(hn, wn_ref[...], preferred_element_type=jnp.float32)
    rst = rst + b_ref[...]
    rst = jnp.where(rst > 0, rst, rst * 0.01)
    o_ref[...] = hh + rst

  return pl.pallas_call(
      body,
      grid=(N_NODES // R,),
      in_specs=[
          pl.BlockSpec((R, D), lambda i: (i, 0)),
          pl.BlockSpec((NC, R, DH), lambda i: (0, i, 0)),
          pl.BlockSpec((NC, R, DW), lambda i: (0, i, 0)),
          pl.BlockSpec((D, D), lambda i: (0, 0)),
          pl.BlockSpec((D, D), lambda i: (0, 0)),
          pl.BlockSpec((1, D), lambda i: (0, 0)),
      ],
      out_specs=pl.BlockSpec((R, D), lambda i: (i, 0)),
      out_shape=jax.ShapeDtypeStruct((N_NODES, D), jnp.float32),
  )(h, acc, deg, W_self, W_neigh, b2)


@jax.jit
def _impl(h, edge_index, W_self, W_neigh, b):
  src = edge_index[0]
  dst = edge_index[1]
  pad = E_PAD - N_EDGES
  src_p = jnp.concatenate([src, jnp.zeros((pad,), jnp.int32)])
  dst_p = jnp.concatenate([dst, jnp.full((pad,), DUMMY, jnp.int32)])
  h2 = h.reshape(2 * N_NODES, DH)
  src0_r = (src_p * 2).reshape(NS, NB, B)
  src1_r = (src_p * 2 + 1).reshape(NS, NB, B)
  dst_r = dst_p.reshape(NS, NB, B)
  acc, deg = _sc_aggregate(h2, src0_r, src1_r, dst_r)
  return _tc_dense(h, acc, deg, W_self, W_neigh, b.reshape(1, D))


def kernel(h, edge_index, W_self, W_neigh, b):
  return _impl(h, edge_index, W_self, W_neigh, b)
